# baseline (speedup 1.0000x reference)
import jax
import jax.numpy as jnp
from jax.experimental import pallas as pl
from jax.experimental.pallas import tpu as pltpu

_B = 4
_S = 8193
_D = 2048
_ROWS = _B * _S
_BLOCK = 256
_NB = (_ROWS + _BLOCK - 1) // _BLOCK
_LAST = _ROWS - (_NB - 1) * _BLOCK
_NSLOTS = 16


def _onehot_body(ids_ref, out_ref, scratch, sems):
    i = pl.program_id(0)
    slot = jax.lax.rem(i, _NSLOTS)

    @pl.when(i >= _NSLOTS)
    def _wait_prev():
        old = i - _NSLOTS
        pltpu.make_async_copy(
            scratch.at[slot],
            out_ref.at[pl.ds(old * _BLOCK, _BLOCK), :],
            sems.at[slot],
        ).wait()

    ids = ids_ref[...]
    iota = jax.lax.broadcasted_iota(jnp.int32, (_BLOCK, _D), 1)
    scratch[slot] = (ids == iota).astype(jnp.float32)

    @pl.when(i < _NB - 1)
    def _copy_full():
        pltpu.make_async_copy(
            scratch.at[slot],
            out_ref.at[pl.ds(i * _BLOCK, _BLOCK), :],
            sems.at[slot],
        ).start()

    @pl.when(i == _NB - 1)
    def _copy_last_and_drain():
        pltpu.make_async_copy(
            scratch.at[slot, pl.ds(0, _LAST), :],
            out_ref.at[pl.ds((_NB - 1) * _BLOCK, _LAST), :],
            sems.at[slot],
        ).start()
        for step in range(max(0, _NB - _NSLOTS), _NB):
            s = step % _NSLOTS
            if step == _NB - 1:
                src = scratch.at[s, pl.ds(0, _LAST), :]
                dst = out_ref.at[pl.ds(step * _BLOCK, _LAST), :]
            else:
                src = scratch.at[s]
                dst = out_ref.at[pl.ds(step * _BLOCK, _BLOCK), :]
            pltpu.make_async_copy(src, dst, sems.at[s]).wait()


def kernel(input_ids):
    padded = jnp.pad(input_ids, ((0, 0), (1, 0)), constant_values=0)
    flat = padded.reshape(-1)
    flat = jnp.pad(flat, (0, _NB * _BLOCK - _ROWS), constant_values=-1)
    ids_col = flat.reshape(_NB * _BLOCK, 1)
    out = pl.pallas_call(
        _onehot_body,
        grid=(_NB,),
        in_specs=[pl.BlockSpec((_BLOCK, 1), lambda i: (i, 0))],
        out_specs=pl.BlockSpec(memory_space=pltpu.MemorySpace.HBM),
        out_shape=jax.ShapeDtypeStruct((_ROWS, _D), jnp.float32),
        scratch_shapes=[
            pltpu.VMEM((_NSLOTS, _BLOCK, _D), jnp.float32),
            pltpu.SemaphoreType.DMA((_NSLOTS,)),
        ],
    )(ids_col)
    return out.reshape(_B, _S, _D)
